# baseline (speedup 1.0000x reference)
import functools

import jax
import jax.numpy as jnp
from jax import lax
from jax.experimental import pallas as pl
from jax.experimental.pallas import tpu as pltpu
from jax.experimental.pallas import tpu_sc as plsc

_NC = 2
_NS = 16
_NW = _NC * _NS


def _mv_body(t_ref, w_ref, o_ref):
    o_ref[...] = jnp.sum(t_ref[...] * w_ref[...], axis=0)


def _table_matvec(tT, w_col, v_pad):
    D, V = tT.shape
    CB = 16384
    return pl.pallas_call(
        _mv_body,
        grid=(pl.cdiv(v_pad, CB),),
        in_specs=[
            pl.BlockSpec((D, CB), lambda g: (0, g)),
            pl.BlockSpec((D, 1), lambda g: (0, 0)),
        ],
        out_specs=pl.BlockSpec((CB,), lambda g: (g,)),
        out_shape=jax.ShapeDtypeStruct((v_pad,), jnp.float32),
    )(tT, w_col)


@functools.cache
def _make_sc_bag(B, L, V):
    BW = B // _NW
    mesh = plsc.VectorSubcoreMesh(core_axis_name="c", subcore_axis_name="s")

    @functools.partial(
        pl.kernel,
        out_type=jax.ShapeDtypeStruct((B,), jnp.float32),
        mesh=mesh,
        compiler_params=pltpu.CompilerParams(needs_layout_passes=False),
        scratch_types=[
            pltpu.VMEM((L, BW), jnp.int32),
            pltpu.VMEM((L, BW), jnp.float32),
            pltpu.VMEM((BW,), jnp.int32),
            pltpu.VMEM((16,), jnp.float32),
            pltpu.VMEM((BW,), jnp.float32),
            pltpu.SemaphoreType.DMA((8,)),
            pltpu.VMEM_SHARED((V,), jnp.float32),
        ],
    )
    def sc_bag(xT_hbm, lens_hbm, tw_hbm, b16_hbm, out_hbm,
               idx_v, vals_v, lens_v, b_v, out_v, sem, tw_sh):
        sid = lax.axis_index("s")
        wid = sid * _NC + lax.axis_index("c")
        base = wid * BW
        pltpu.sync_copy(xT_hbm.at[:, pl.ds(base, BW)], idx_v)
        pltpu.sync_copy(lens_hbm.at[pl.ds(base, BW)], lens_v)
        pltpu.sync_copy(b16_hbm, b_v)
        CHUNK = 65536

        @pl.when(sid < _NS - 1)
        def _():
            pltpu.sync_copy(tw_hbm.at[pl.ds(sid * CHUNK, CHUNK)],
                            tw_sh.at[pl.ds(sid * CHUNK, CHUNK)])

        @pl.when(sid == _NS - 1)
        def _():
            rem = V - (_NS - 1) * CHUNK
            pltpu.sync_copy(tw_hbm.at[pl.ds((_NS - 1) * CHUNK, rem)],
                            tw_sh.at[pl.ds((_NS - 1) * CHUNK, rem)])

        plsc.subcore_barrier()
        K = 8
        NG = BW // 16
        lens_gs = [lens_v[pl.ds(g * 16, 16)] for g in range(NG)]

        def prime(c, carry):
            pltpu.async_copy(tw_sh.at[idx_v.at[c]], vals_v.at[c], sem.at[c])
            return carry

        lax.fori_loop(0, K, prime, 0)

        def step(j, accs):
            @pl.when(j + K < L)
            def _():
                c = j + K
                pltpu.async_copy(tw_sh.at[idx_v.at[c]], vals_v.at[c],
                                 sem.at[lax.rem(c, K)])

            pltpu.make_async_copy(tw_sh.at[idx_v.at[j]], vals_v.at[j],
                                  sem.at[lax.rem(j, K)]).wait()
            out = []
            for g in range(NG):
                v = vals_v[j, pl.ds(g * 16, 16)]
                out.append(accs[g] + jnp.where(j < lens_gs[g], v, 0.0))
            return tuple(out)

        accs = lax.fori_loop(
            0, L, step, tuple(jnp.zeros((16,), jnp.float32) for _ in range(NG)))

        bias = b_v[...]
        for g in range(NG):
            sl = pl.ds(g * 16, 16)
            out_v[sl] = accs[g] / lens_gs[g].astype(jnp.float32) + bias
        pltpu.sync_copy(out_v, out_hbm.at[pl.ds(base, BW)])

    return sc_bag


def kernel(x, lens, table, W, b):
    B, L = x.shape
    V, D = table.shape
    v_pad = ((V + 1023) // 1024) * 1024
    tw = _table_matvec(table.T, W.T, v_pad)
    b16 = jnp.broadcast_to(b.reshape(1).astype(jnp.float32), (16,))
    return _make_sc_bag(B, L, v_pad)(x.T, lens, tw, b16)


# speedup vs baseline: 1.1177x; 1.1177x over previous
import functools

import jax
import jax.numpy as jnp
from jax import lax
from jax.experimental import pallas as pl
from jax.experimental.pallas import tpu as pltpu
from jax.experimental.pallas import tpu_sc as plsc

_NC = 2
_NS = 16
_NW = _NC * _NS


def _mv_body(t_ref, w_ref, o_ref):
    o_ref[...] = jnp.sum(t_ref[...] * w_ref[...], axis=0)


def _table_matvec(tT, w_col, v_pad):
    D, V = tT.shape
    CB = 49152
    return pl.pallas_call(
        _mv_body,
        grid=(pl.cdiv(v_pad, CB),),
        in_specs=[
            pl.BlockSpec((D, CB), lambda g: (0, g)),
            pl.BlockSpec((D, 1), lambda g: (0, 0)),
        ],
        out_specs=pl.BlockSpec((CB,), lambda g: (g,)),
        out_shape=jax.ShapeDtypeStruct((v_pad,), jnp.float32),
    )(tT, w_col)


@functools.cache
def _make_sc_bag(B, L, V):
    BW = B // _NW
    mesh = plsc.VectorSubcoreMesh(core_axis_name="c", subcore_axis_name="s")

    @functools.partial(
        pl.kernel,
        out_type=jax.ShapeDtypeStruct((B,), jnp.float32),
        mesh=mesh,
        compiler_params=pltpu.CompilerParams(needs_layout_passes=False),
        scratch_types=[
            pltpu.VMEM((L, BW), jnp.int32),
            pltpu.VMEM((L, BW), jnp.float32),
            pltpu.VMEM((BW,), jnp.int32),
            pltpu.VMEM((16,), jnp.float32),
            pltpu.VMEM((BW,), jnp.float32),
            pltpu.SemaphoreType.DMA((8,)),
            pltpu.VMEM_SHARED((V,), jnp.float32),
        ],
    )
    def sc_bag(xT_hbm, lens_hbm, tw_hbm, b16_hbm, out_hbm,
               idx_v, vals_v, lens_v, b_v, out_v, sem, tw_sh):
        sid = lax.axis_index("s")
        wid = sid * _NC + lax.axis_index("c")
        base = wid * BW
        pltpu.sync_copy(xT_hbm.at[:, pl.ds(base, BW)], idx_v)
        pltpu.sync_copy(lens_hbm.at[pl.ds(base, BW)], lens_v)
        pltpu.sync_copy(b16_hbm, b_v)
        CHUNK = 65536

        @pl.when(sid < _NS - 1)
        def _():
            pltpu.sync_copy(tw_hbm.at[pl.ds(sid * CHUNK, CHUNK)],
                            tw_sh.at[pl.ds(sid * CHUNK, CHUNK)])

        @pl.when(sid == _NS - 1)
        def _():
            rem = V - (_NS - 1) * CHUNK
            pltpu.sync_copy(tw_hbm.at[pl.ds((_NS - 1) * CHUNK, rem)],
                            tw_sh.at[pl.ds((_NS - 1) * CHUNK, rem)])

        plsc.subcore_barrier()
        K = 8
        NG = BW // 16
        lens_gs = [lens_v[pl.ds(g * 16, 16)] for g in range(NG)]

        def prime(c, carry):
            pltpu.async_copy(tw_sh.at[idx_v.at[c]], vals_v.at[c], sem.at[c])
            return carry

        lax.fori_loop(0, K, prime, 0)

        def step(j, accs):
            @pl.when(j + K < L)
            def _():
                c = j + K
                pltpu.async_copy(tw_sh.at[idx_v.at[c]], vals_v.at[c],
                                 sem.at[lax.rem(c, K)])

            pltpu.make_async_copy(tw_sh.at[idx_v.at[j]], vals_v.at[j],
                                  sem.at[lax.rem(j, K)]).wait()
            out = []
            for g in range(NG):
                v = vals_v[j, pl.ds(g * 16, 16)]
                out.append(accs[g] + jnp.where(j < lens_gs[g], v, 0.0))
            return tuple(out)

        accs = lax.fori_loop(
            0, L, step, tuple(jnp.zeros((16,), jnp.float32) for _ in range(NG)))

        bias = b_v[...]
        for g in range(NG):
            sl = pl.ds(g * 16, 16)
            out_v[sl] = accs[g] / lens_gs[g].astype(jnp.float32) + bias
        pltpu.sync_copy(out_v, out_hbm.at[pl.ds(base, BW)])

    return sc_bag


def kernel(x, lens, table, W, b):
    B, L = x.shape
    V, D = table.shape
    v_pad = ((V + 1023) // 1024) * 1024
    tw = _table_matvec(table.T, W.T, v_pad)
    b16 = jnp.broadcast_to(b.reshape(1).astype(jnp.float32), (16,))
    return _make_sc_bag(B, L, v_pad)(x.T, lens, tw, b16)


# speedup vs baseline: 1.1598x; 1.0377x over previous
import functools

import jax
import jax.numpy as jnp
from jax import lax
from jax.experimental import pallas as pl
from jax.experimental.pallas import tpu as pltpu
from jax.experimental.pallas import tpu_sc as plsc

_NC = 2
_NS = 16
_NW = _NC * _NS


def _mv_body(t_ref, w_ref, o_ref):
    o_ref[...] = jnp.sum(t_ref[...] * w_ref[...], axis=0)


def _table_matvec(tT, w_col, v_pad):
    D, V = tT.shape
    CB = 32768
    return pl.pallas_call(
        _mv_body,
        grid=(pl.cdiv(v_pad, CB),),
        in_specs=[
            pl.BlockSpec((D, CB), lambda g: (0, g)),
            pl.BlockSpec((D, 1), lambda g: (0, 0)),
        ],
        out_specs=pl.BlockSpec((CB,), lambda g: (g,)),
        out_shape=jax.ShapeDtypeStruct((v_pad,), jnp.float32),
    )(tT, w_col)


@functools.cache
def _make_sc_bag(B, L, V):
    BW = B // _NW
    mesh = plsc.VectorSubcoreMesh(core_axis_name="c", subcore_axis_name="s")

    @functools.partial(
        pl.kernel,
        out_type=jax.ShapeDtypeStruct((B,), jnp.float32),
        mesh=mesh,
        compiler_params=pltpu.CompilerParams(needs_layout_passes=False),
        scratch_types=[
            pltpu.VMEM((L, BW), jnp.int32),
            pltpu.VMEM((L, BW), jnp.float32),
            pltpu.VMEM((BW,), jnp.int32),
            pltpu.VMEM((16,), jnp.float32),
            pltpu.VMEM((BW,), jnp.float32),
            pltpu.SemaphoreType.DMA((8,)),
            pltpu.SemaphoreType.DMA,
            pltpu.SemaphoreType.DMA,
            pltpu.VMEM_SHARED((V,), jnp.float32),
        ],
    )
    def sc_bag(xT_hbm, lens_hbm, tw_hbm, b16_hbm, out_hbm,
               idx_v, vals_v, lens_v, b_v, out_v, sem, sem_x, sem_lb, tw_sh):
        sid = lax.axis_index("s")
        wid = sid * _NC + lax.axis_index("c")
        base = wid * BW
        cx = pltpu.async_copy(xT_hbm.at[:, pl.ds(base, BW)], idx_v, sem_x)
        cl = pltpu.async_copy(lens_hbm.at[pl.ds(base, BW)], lens_v, sem_lb)
        cb = pltpu.async_copy(b16_hbm, b_v, sem_lb)
        CHUNK = 65536

        @pl.when(sid < _NS - 1)
        def _():
            pltpu.sync_copy(tw_hbm.at[pl.ds(sid * CHUNK, CHUNK)],
                            tw_sh.at[pl.ds(sid * CHUNK, CHUNK)])

        @pl.when(sid == _NS - 1)
        def _():
            rem = V - (_NS - 1) * CHUNK
            pltpu.sync_copy(tw_hbm.at[pl.ds((_NS - 1) * CHUNK, rem)],
                            tw_sh.at[pl.ds((_NS - 1) * CHUNK, rem)])

        cx.wait()
        plsc.subcore_barrier()
        cl.wait()
        cb.wait()
        K = 8
        NG = BW // 16
        lens_gs = [lens_v[pl.ds(g * 16, 16)] for g in range(NG)]

        def prime(c, carry):
            pltpu.async_copy(tw_sh.at[idx_v.at[c]], vals_v.at[c], sem.at[c])
            return carry

        lax.fori_loop(0, K, prime, 0)

        def step(j, accs):
            @pl.when(j + K < L)
            def _():
                c = j + K
                pltpu.async_copy(tw_sh.at[idx_v.at[c]], vals_v.at[c],
                                 sem.at[lax.rem(c, K)])

            pltpu.make_async_copy(tw_sh.at[idx_v.at[j]], vals_v.at[j],
                                  sem.at[lax.rem(j, K)]).wait()
            out = []
            for g in range(NG):
                v = vals_v[j, pl.ds(g * 16, 16)]
                out.append(accs[g] + jnp.where(j < lens_gs[g], v, 0.0))
            return tuple(out)

        accs = lax.fori_loop(
            0, L, step, tuple(jnp.zeros((16,), jnp.float32) for _ in range(NG)))

        bias = b_v[...]
        for g in range(NG):
            sl = pl.ds(g * 16, 16)
            out_v[sl] = accs[g] / lens_gs[g].astype(jnp.float32) + bias
        pltpu.sync_copy(out_v, out_hbm.at[pl.ds(base, BW)])

    return sc_bag


def kernel(x, lens, table, W, b):
    B, L = x.shape
    V, D = table.shape
    v_pad = ((V + 1023) // 1024) * 1024
    tw = _table_matvec(table.T, W.T, v_pad)
    b16 = jnp.broadcast_to(b.reshape(1).astype(jnp.float32), (16,))
    return _make_sc_bag(B, L, v_pad)(x.T, lens, tw, b16)
